# baseline (speedup 1.0000x reference)
import functools

import jax
import jax.numpy as jnp
from jax import lax
from jax.experimental import pallas as pl
from jax.experimental.pallas import tpu as pltpu
from jax.experimental.pallas import tpu_sc as plsc

_VOCAB = 1000
_D = 1000
_BATCH = 4096
_SEQ = 20
_B = _BATCH * _SEQ
_NW = 32
_BPW = _B // _NW
_C = 40
_CB = _C // _SEQ
_NBUF = 2
_NCHUNK = _BPW // _C

_mesh = plsc.VectorSubcoreMesh(core_axis_name="c", subcore_axis_name="s")


@functools.partial(
    pl.kernel,
    mesh=_mesh,
    out_type=jax.ShapeDtypeStruct((_BATCH, _SEQ, _D), jnp.float32),
    scratch_types=[
        pltpu.VMEM((_BPW,), jnp.int32),
        [pltpu.VMEM((_C, _D), jnp.float32) for _ in range(_NBUF)],
        [pltpu.SemaphoreType.DMA for _ in range(_NBUF)],
        [pltpu.SemaphoreType.DMA for _ in range(_NBUF)],
    ],
    compiler_params=pltpu.CompilerParams(use_tc_tiling_on_sc=False),
)
def _emb_gather(idx_hbm, table_hbm, out_hbm, idx_v, rows, gsem, wsem):
    wid = lax.axis_index("s") * 2 + lax.axis_index("c")
    base = wid * _BPW
    bbase = wid * (_BPW // _SEQ)
    pltpu.sync_copy(idx_hbm.at[pl.ds(base, _BPW)], idx_v)

    def gather(g, b):
        pltpu.async_copy(
            table_hbm.at[idx_v.at[pl.ds(g * _C, _C)]], rows[b], gsem[b]
        )

    def gather_wait(b):
        pltpu.make_async_copy(
            table_hbm.at[idx_v.at[pl.ds(0, _C)]], rows[b], gsem[b]
        ).wait()

    def writeback(g, b):
        for j in range(_CB):
            pltpu.async_copy(
                rows[b].at[pl.ds(j * _SEQ, _SEQ)],
                out_hbm.at[bbase + g * _CB + j],
                wsem[b],
            )

    def writeback_wait(b):
        for j in range(_CB):
            pltpu.make_async_copy(
                rows[b].at[pl.ds(j * _SEQ, _SEQ)],
                out_hbm.at[bbase],
                wsem[b],
            ).wait()

    for b in range(_NBUF):
        gather(b, b)

    @pl.loop(0, _NCHUNK - _NBUF, step=_NBUF)
    def _round(c):
        for b in range(_NBUF):
            g = c + b
            gather_wait(b)
            writeback(g, b)
            writeback_wait(b)
            gather(g + _NBUF, b)

    for b in range(_NBUF):
        g = _NCHUNK - _NBUF + b
        gather_wait(b)
        writeback(g, b)
        writeback_wait(b)


def kernel(x, table):
    flat = x.reshape(-1).astype(jnp.int32)
    return _emb_gather(flat, table)
